# baseline (speedup 1.0000x reference)
import functools

import jax
import jax.numpy as jnp
from jax import lax
from jax.experimental import pallas as pl
from jax.experimental.pallas import tpu as pltpu
from jax.experimental.pallas import tpu_sc as plsc

K = 32
R = 256
D = 16
SC_CHUNK = 256
PADW = 128


STRIPS = 32


def _topk_body(ct_ref, x_ref, sqc_ref, sqr_ref, idx_ref, d_ref, *, n, r):
    w = n // STRIPS
    big = jnp.int32(1 << 30)
    inf = jnp.float32(jnp.inf)
    ct = ct_ref[...]
    x = x_ref[...]
    sq = sqc_ref[...]
    sqb = sqr_ref[...]
    d = sqb + sq - 2.0 * jnp.dot(x, ct, preferred_element_type=jnp.float32)
    d_ref[...] = d

    lanes_w = jax.lax.broadcasted_iota(jnp.int32, (r, w), 1)
    cols = jax.lax.broadcasted_iota(jnp.int32, (r, K), 1)

    acc = d_ref[:, 0:w]
    amt = jnp.zeros((r, w), jnp.int32)
    for t in range(1, STRIPS):
        s = d_ref[:, t * w:(t + 1) * w]
        lt = s < acc
        amt = jnp.where(lt, t, amt)
        acc = jnp.where(lt, s, acc)
    agidx = amt * w + lanes_w

    def bstep(k, carry):
        acc, selj = carry
        m = jnp.min(acc, axis=1, keepdims=True)
        g0 = jnp.min(jnp.where(acc == m, agidx, big), axis=1, keepdims=True)
        jc = g0 % w
        selj = jnp.where(cols == k, jc, selj)
        acc = jnp.where(lanes_w == jc, inf, acc)
        return acc, selj

    _, selj = jax.lax.fori_loop(
        0, K, bstep, (acc, jnp.zeros((r, K), jnp.int32)), unroll=False)

    lane_k = jax.lax.broadcasted_iota(jnp.int32, (r, K), 1)
    for kk in (2, 4, 8, 16, 32):
        jj = kk // 2
        while jj >= 1:
            partner = jnp.take_along_axis(selj, lane_k ^ jj, axis=1)
            up = (lane_k & kk) == 0
            keep_min = ((lane_k & jj) == 0) == up
            selj = jnp.where(keep_min, jnp.minimum(selj, partner),
                             jnp.maximum(selj, partner))
            jj //= 2

    n_half = max(w // 128, 1)
    jl = selj % min(w, 128)
    jh = selj // min(w, 128)
    cand_parts, gidx_parts = [], []
    for t in range(STRIPS):
        v = None
        for h in range(n_half):
            hw = min(w, 128)
            part = d_ref[:, t * w + h * hw: t * w + (h + 1) * hw]
            g = jnp.take_along_axis(part, jl, axis=1)
            v = g if v is None else jnp.where(jh == h, g, v)
        cand_parts.append(v)
        gidx_parts.append(t * w + selj)
    cand = jnp.concatenate(cand_parts, axis=1)
    gidx = jnp.concatenate(gidx_parts, axis=1)

    lanes_c = jax.lax.broadcasted_iota(jnp.int32, (r, K * STRIPS), 1)

    def dstep(k, cand):
        p = jnp.argmin(cand, axis=1).reshape(r, 1).astype(jnp.int32)
        ph = p // 128
        pl_ = p % 128
        gwin = None
        for h in range(K * STRIPS // 128):
            g = jnp.take_along_axis(gidx[:, h * 128:(h + 1) * 128], pl_,
                                    axis=1)
            gwin = g if gwin is None else jnp.where(ph == h, g, gwin)
        idx_ref[...] = jnp.where(cols == k, gwin, idx_ref[...])
        return jnp.where(lanes_c == p, inf, cand)

    jax.lax.fori_loop(0, K, dstep, cand, unroll=False)


def _nbr_topk(coords):
    n, d_feat = coords.shape
    r = R if n % R == 0 else n
    grid = (n // r,)
    body = functools.partial(_topk_body, n=n, r=r)
    sq = jnp.sum(coords * coords, axis=1)
    return pl.pallas_call(
        body,
        grid=grid,
        in_specs=[
            pl.BlockSpec((d_feat, n), lambda i: (0, 0)),
            pl.BlockSpec((r, d_feat), lambda i: (i, 0)),
            pl.BlockSpec((1, n), lambda i: (0, 0)),
            pl.BlockSpec((r, 1), lambda i: (i, 0)),
        ],
        out_specs=pl.BlockSpec((r, K), lambda i: (i, 0)),
        out_shape=jax.ShapeDtypeStruct((n, K), jnp.int32),
        scratch_shapes=[pltpu.VMEM((r, n), jnp.float32)],
    )(coords.T, coords, sq.reshape(1, n), sq.reshape(n, 1))


def _newton_sqrt(x):
    i = lax.bitcast_convert_type(x, jnp.int32)
    y = lax.bitcast_convert_type(
        jnp.int32(0x1FBD1DF5) + lax.shift_right_logical(i, 1), jnp.float32)
    for _ in range(3):
        y = 0.5 * (y + x / y)
    return y


def _edge_weights(coords, src):
    n = coords.shape[0]
    e_total = n * K
    nw = 32
    per = e_total // nw
    n_chunks = per // SC_CHUNK
    coords_pad = jnp.pad(coords, ((0, 0), (0, PADW - D)))
    mesh = plsc.VectorSubcoreMesh(core_axis_name="c", subcore_axis_name="s")

    @functools.partial(
        pl.kernel, mesh=mesh,
        out_type=jax.ShapeDtypeStruct((e_total,), jnp.float32),
        scratch_types=[
            pltpu.VMEM((SC_CHUNK,), jnp.int32),
            pltpu.VMEM((SC_CHUNK, PADW), jnp.float32),
            pltpu.VMEM((per // K, D), jnp.float32),
            pltpu.VMEM((SC_CHUNK,), jnp.float32),
            pltpu.SemaphoreType.DMA,
        ],
    )
    def edge_kernel(coords_hbm, cpad_hbm, src_hbm, out_hbm, idx_v, rows_v,
                    dst_v, w_v, sem):
        lane = lax.iota(jnp.int32, D)
        wid = lax.axis_index("s") * 2 + lax.axis_index("c")
        base = wid * per
        row0 = pl.multiple_of(base // K, per // K)
        pltpu.sync_copy(coords_hbm.at[pl.ds(row0, per // K)], dst_v)

        def chunk(ci, _):
            off = pl.multiple_of(base + ci * SC_CHUNK, SC_CHUNK)
            pltpu.sync_copy(src_hbm.at[pl.ds(off, SC_CHUNK)], idx_v)
            for j in range(SC_CHUNK // 128):
                pltpu.async_copy(
                    cpad_hbm.at[idx_v.at[pl.ds(j * 128, 128)]],
                    rows_v.at[pl.ds(j * 128, 128)], sem)
            for j in range(SC_CHUNK // 128):
                pltpu.make_async_copy(
                    cpad_hbm.at[idx_v.at[pl.ds(j * 128, 128)]],
                    rows_v.at[pl.ds(j * 128, 128)], sem).wait()

            def group(g, _):
                dv = dst_v[ci * (SC_CHUNK // K) + g // 2]
                acc = jnp.zeros((D,), jnp.float32)
                for i in range(D):
                    diff = rows_v[g * D + i, pl.ds(0, D)] - dv
                    s = diff * diff
                    for sh in (1, 2, 4, 8):
                        s = s + jnp.take(s, lane ^ sh)
                    acc = jnp.where(lane == i, s, acc)
                w_v[pl.ds(g * D, D)] = _newton_sqrt(acc)
                return 0

            lax.fori_loop(0, SC_CHUNK // D, group, 0, unroll=False)
            pltpu.sync_copy(w_v, out_hbm.at[pl.ds(off, SC_CHUNK)])
            return 0

        lax.fori_loop(0, n_chunks, chunk, 0, unroll=False)

    return edge_kernel(coords, coords_pad, src)


def kernel(coords):
    n, _ = coords.shape
    nbr = _nbr_topk(coords)
    src = nbr.reshape(-1)
    dst = jnp.repeat(jnp.arange(n, dtype=jnp.int32), K)
    edge_idx = jnp.stack([src, dst], axis=0)
    edge_weights = _edge_weights(coords, src)
    return edge_idx, edge_weights


# speedup vs baseline: 1.7716x; 1.7716x over previous
import functools

import jax
import jax.numpy as jnp
from jax import lax
from jax.experimental import pallas as pl
from jax.experimental.pallas import tpu as pltpu
from jax.experimental.pallas import tpu_sc as plsc

K = 32
R = 256
D = 16
SC_CHUNK = 256
PADW = 128


STRIPS = 16


def _topk_body(ct_ref, x_ref, sqc_ref, sqr_ref, idx_ref, d_ref, *, n, r):
    w = n // STRIPS
    big = jnp.int32(1 << 30)
    inf = jnp.float32(jnp.inf)
    ct = ct_ref[...]
    x = x_ref[...]
    sq = sqc_ref[...]
    sqb = sqr_ref[...]
    d = sqb + sq - 2.0 * jnp.dot(x, ct, preferred_element_type=jnp.float32)
    d_ref[...] = d

    lanes_w = jax.lax.broadcasted_iota(jnp.int32, (r, w), 1)
    cols = jax.lax.broadcasted_iota(jnp.int32, (r, K), 1)

    acc = d_ref[:, 0:w]
    amt = jnp.zeros((r, w), jnp.int32)
    for t in range(1, STRIPS):
        s = d_ref[:, t * w:(t + 1) * w]
        lt = s < acc
        amt = jnp.where(lt, t, amt)
        acc = jnp.where(lt, s, acc)
    agidx = amt * w + lanes_w

    def bstep(k, carry):
        acc, selj = carry
        m = jnp.min(acc, axis=1, keepdims=True)
        g0 = jnp.min(jnp.where(acc == m, agidx, big), axis=1, keepdims=True)
        jc = g0 % w
        selj = jnp.where(cols == k, jc, selj)
        acc = jnp.where(lanes_w == jc, inf, acc)
        return acc, selj

    _, selj = jax.lax.fori_loop(
        0, K, bstep, (acc, jnp.zeros((r, K), jnp.int32)), unroll=False)

    n_half = max(w // 128, 1)
    jl = selj % min(w, 128)
    jh = selj // min(w, 128)
    cand_parts, gidx_parts = [], []
    for t in range(STRIPS):
        v = None
        for h in range(n_half):
            hw = min(w, 128)
            part = d_ref[:, t * w + h * hw: t * w + (h + 1) * hw]
            g = jnp.take_along_axis(part, jl, axis=1)
            v = g if v is None else jnp.where(jh == h, g, v)
        cand_parts.append(v)
        gidx_parts.append(t * w + selj)
    cand = jnp.concatenate(cand_parts, axis=1)
    gidx = jnp.concatenate(gidx_parts, axis=1)

    def dstep(k, cand):
        m = jnp.min(cand, axis=1, keepdims=True)
        gwin = jnp.min(jnp.where(cand == m, gidx, big), axis=1, keepdims=True)
        idx_ref[...] = jnp.where(cols == k, gwin, idx_ref[...])
        return jnp.where((cand == m) & (gidx == gwin), inf, cand)

    jax.lax.fori_loop(0, K, dstep, cand, unroll=False)


def _nbr_topk(coords):
    n, d_feat = coords.shape
    r = R if n % R == 0 else n
    grid = (n // r,)
    body = functools.partial(_topk_body, n=n, r=r)
    sq = jnp.sum(coords * coords, axis=1)
    return pl.pallas_call(
        body,
        grid=grid,
        in_specs=[
            pl.BlockSpec((d_feat, n), lambda i: (0, 0)),
            pl.BlockSpec((r, d_feat), lambda i: (i, 0)),
            pl.BlockSpec((1, n), lambda i: (0, 0)),
            pl.BlockSpec((r, 1), lambda i: (i, 0)),
        ],
        out_specs=pl.BlockSpec((r, K), lambda i: (i, 0)),
        out_shape=jax.ShapeDtypeStruct((n, K), jnp.int32),
        scratch_shapes=[pltpu.VMEM((r, n), jnp.float32)],
    )(coords.T, coords, sq.reshape(1, n), sq.reshape(n, 1))


def _newton_sqrt(x):
    i = lax.bitcast_convert_type(x, jnp.int32)
    y = lax.bitcast_convert_type(
        jnp.int32(0x1FBD1DF5) + lax.shift_right_logical(i, 1), jnp.float32)
    for _ in range(3):
        y = 0.5 * (y + x / y)
    return y


def _edge_weights(coords, src):
    n = coords.shape[0]
    e_total = n * K
    nw = 32
    per = e_total // nw
    n_chunks = per // SC_CHUNK
    coords_pad = jnp.pad(coords, ((0, 0), (0, PADW - D)))
    mesh = plsc.VectorSubcoreMesh(core_axis_name="c", subcore_axis_name="s")

    @functools.partial(
        pl.kernel, mesh=mesh,
        out_type=jax.ShapeDtypeStruct((e_total,), jnp.float32),
        scratch_types=[
            pltpu.VMEM((SC_CHUNK,), jnp.int32),
            pltpu.VMEM((SC_CHUNK, PADW), jnp.float32),
            pltpu.VMEM((per // K, D), jnp.float32),
            pltpu.VMEM((SC_CHUNK,), jnp.float32),
            pltpu.SemaphoreType.DMA,
        ],
    )
    def edge_kernel(coords_hbm, cpad_hbm, src_hbm, out_hbm, idx_v, rows_v,
                    dst_v, w_v, sem):
        lane = lax.iota(jnp.int32, D)
        wid = lax.axis_index("s") * 2 + lax.axis_index("c")
        base = wid * per
        row0 = pl.multiple_of(base // K, per // K)
        pltpu.sync_copy(coords_hbm.at[pl.ds(row0, per // K)], dst_v)

        def chunk(ci, _):
            off = pl.multiple_of(base + ci * SC_CHUNK, SC_CHUNK)
            pltpu.sync_copy(src_hbm.at[pl.ds(off, SC_CHUNK)], idx_v)
            for j in range(SC_CHUNK // 128):
                pltpu.async_copy(
                    cpad_hbm.at[idx_v.at[pl.ds(j * 128, 128)]],
                    rows_v.at[pl.ds(j * 128, 128)], sem)
            for j in range(SC_CHUNK // 128):
                pltpu.make_async_copy(
                    cpad_hbm.at[idx_v.at[pl.ds(j * 128, 128)]],
                    rows_v.at[pl.ds(j * 128, 128)], sem).wait()

            def group(g, _):
                dv = dst_v[ci * (SC_CHUNK // K) + g // 2]
                acc = jnp.zeros((D,), jnp.float32)
                for i in range(D):
                    diff = rows_v[g * D + i, pl.ds(0, D)] - dv
                    s = diff * diff
                    for sh in (1, 2, 4, 8):
                        s = s + jnp.take(s, lane ^ sh)
                    acc = jnp.where(lane == i, s, acc)
                w_v[pl.ds(g * D, D)] = _newton_sqrt(acc)
                return 0

            lax.fori_loop(0, SC_CHUNK // D, group, 0, unroll=False)
            pltpu.sync_copy(w_v, out_hbm.at[pl.ds(off, SC_CHUNK)])
            return 0

        lax.fori_loop(0, n_chunks, chunk, 0, unroll=False)

    return edge_kernel(coords, coords_pad, src)


def kernel(coords):
    n, _ = coords.shape
    nbr = _nbr_topk(coords)
    src = nbr.reshape(-1)
    dst = jnp.repeat(jnp.arange(n, dtype=jnp.int32), K)
    edge_idx = jnp.stack([src, dst], axis=0)
    edge_weights = _edge_weights(coords, src)
    return edge_idx, edge_weights


# speedup vs baseline: 1.8553x; 1.0473x over previous
import functools

import jax
import jax.numpy as jnp
from jax import lax
from jax.experimental import pallas as pl
from jax.experimental.pallas import tpu as pltpu
from jax.experimental.pallas import tpu_sc as plsc

K = 32
R = 256
D = 16
SC_CHUNK = 256
PADW = 128


STRIPS = 16


def _topk_body(ct_ref, x_ref, sqc_ref, sqr_ref, idx_ref, d_ref, *, n, r):
    w = n // STRIPS
    big = jnp.int32(1 << 30)
    inf = jnp.float32(jnp.inf)
    ct = ct_ref[...]
    x = x_ref[...]
    sq = sqc_ref[...]
    sqb = sqr_ref[...]
    d = sqb + sq - 2.0 * jnp.dot(x, ct, preferred_element_type=jnp.float32)
    d_ref[...] = d

    lanes_w = jax.lax.broadcasted_iota(jnp.int32, (r, w), 1)
    cols = jax.lax.broadcasted_iota(jnp.int32, (r, K), 1)

    acc = d_ref[:, 0:w]
    amt = jnp.zeros((r, w), jnp.int32)
    for t in range(1, STRIPS):
        s = d_ref[:, t * w:(t + 1) * w]
        lt = s < acc
        amt = jnp.where(lt, t, amt)
        acc = jnp.where(lt, s, acc)
    agidx = amt * w + lanes_w

    def bstep(k, carry):
        acc, selj = carry
        m = jnp.min(acc, axis=1, keepdims=True)
        eqm = acc == m
        g0 = jnp.min(jnp.where(eqm, agidx, big), axis=1, keepdims=True)
        jc = g0 % w
        selj = jnp.where(cols == k, jc, selj)
        acc = jnp.where(lanes_w == jc, inf, acc)
        return acc, selj

    _, selj = jax.lax.fori_loop(
        0, K, bstep, (acc, jnp.zeros((r, K), jnp.int32)), unroll=False)

    n_half = max(w // 128, 1)
    jl = selj % min(w, 128)
    jh = selj // min(w, 128)
    cand_parts, gidx_parts = [], []
    for t in range(STRIPS):
        v = None
        for h in range(n_half):
            hw = min(w, 128)
            part = d_ref[:, t * w + h * hw: t * w + (h + 1) * hw]
            g = jnp.take_along_axis(part, jl, axis=1)
            v = g if v is None else jnp.where(jh == h, g, v)
        cand_parts.append(v)
        gidx_parts.append(t * w + selj)
    cand = jnp.concatenate(cand_parts, axis=1)
    gidx = jnp.concatenate(gidx_parts, axis=1)

    def dstep(k, cand):
        m = jnp.min(cand, axis=1, keepdims=True)
        eqm = cand == m
        masked_g = jnp.where(eqm, gidx, big)
        gwin = jnp.min(masked_g, axis=1, keepdims=True)
        idx_ref[...] = jnp.where(cols == k, gwin, idx_ref[...])
        return jnp.where(masked_g == gwin, inf, cand)

    jax.lax.fori_loop(0, K, dstep, cand, unroll=False)


def _nbr_topk(coords):
    n, d_feat = coords.shape
    r = R if n % R == 0 else n
    grid = (n // r,)
    body = functools.partial(_topk_body, n=n, r=r)
    sq = jnp.sum(coords * coords, axis=1)
    return pl.pallas_call(
        body,
        grid=grid,
        in_specs=[
            pl.BlockSpec((d_feat, n), lambda i: (0, 0)),
            pl.BlockSpec((r, d_feat), lambda i: (i, 0)),
            pl.BlockSpec((1, n), lambda i: (0, 0)),
            pl.BlockSpec((r, 1), lambda i: (i, 0)),
        ],
        out_specs=pl.BlockSpec((r, K), lambda i: (i, 0)),
        out_shape=jax.ShapeDtypeStruct((n, K), jnp.int32),
        scratch_shapes=[pltpu.VMEM((r, n), jnp.float32)],
    )(coords.T, coords, sq.reshape(1, n), sq.reshape(n, 1))


def _newton_sqrt(x):
    i = lax.bitcast_convert_type(x, jnp.int32)
    y = lax.bitcast_convert_type(
        jnp.int32(0x1FBD1DF5) + lax.shift_right_logical(i, 1), jnp.float32)
    for _ in range(3):
        y = 0.5 * (y + x / y)
    return y


def _edge_weights(coords, src):
    n = coords.shape[0]
    e_total = n * K
    nw = 32
    per = e_total // nw
    n_chunks = per // SC_CHUNK
    coords_pad = jnp.pad(coords, ((0, 0), (0, PADW - D)))
    mesh = plsc.VectorSubcoreMesh(core_axis_name="c", subcore_axis_name="s")

    @functools.partial(
        pl.kernel, mesh=mesh,
        out_type=jax.ShapeDtypeStruct((e_total,), jnp.float32),
        scratch_types=[
            pltpu.VMEM((SC_CHUNK,), jnp.int32),
            pltpu.VMEM((SC_CHUNK, PADW), jnp.float32),
            pltpu.VMEM((per // K, D), jnp.float32),
            pltpu.VMEM((SC_CHUNK,), jnp.float32),
            pltpu.SemaphoreType.DMA,
        ],
    )
    def edge_kernel(coords_hbm, cpad_hbm, src_hbm, out_hbm, idx_v, rows_v,
                    dst_v, w_v, sem):
        lane = lax.iota(jnp.int32, D)
        wid = lax.axis_index("s") * 2 + lax.axis_index("c")
        base = wid * per
        row0 = pl.multiple_of(base // K, per // K)
        pltpu.sync_copy(coords_hbm.at[pl.ds(row0, per // K)], dst_v)

        def chunk(ci, _):
            off = pl.multiple_of(base + ci * SC_CHUNK, SC_CHUNK)
            pltpu.sync_copy(src_hbm.at[pl.ds(off, SC_CHUNK)], idx_v)
            for j in range(SC_CHUNK // 128):
                pltpu.async_copy(
                    cpad_hbm.at[idx_v.at[pl.ds(j * 128, 128)]],
                    rows_v.at[pl.ds(j * 128, 128)], sem)
            for j in range(SC_CHUNK // 128):
                pltpu.make_async_copy(
                    cpad_hbm.at[idx_v.at[pl.ds(j * 128, 128)]],
                    rows_v.at[pl.ds(j * 128, 128)], sem).wait()

            def group(g, _):
                dv = dst_v[ci * (SC_CHUNK // K) + g // 2]
                acc = jnp.zeros((D,), jnp.float32)
                for i in range(D):
                    diff = rows_v[g * D + i, pl.ds(0, D)] - dv
                    s = diff * diff
                    for sh in (1, 2, 4, 8):
                        s = s + jnp.take(s, lane ^ sh)
                    acc = jnp.where(lane == i, s, acc)
                w_v[pl.ds(g * D, D)] = _newton_sqrt(acc)
                return 0

            lax.fori_loop(0, SC_CHUNK // D, group, 0, unroll=False)
            pltpu.sync_copy(w_v, out_hbm.at[pl.ds(off, SC_CHUNK)])
            return 0

        lax.fori_loop(0, n_chunks, chunk, 0, unroll=False)

    return edge_kernel(coords, coords_pad, src)


def kernel(coords):
    n, _ = coords.shape
    nbr = _nbr_topk(coords)
    src = nbr.reshape(-1)
    dst = jnp.repeat(jnp.arange(n, dtype=jnp.int32), K)
    edge_idx = jnp.stack([src, dst], axis=0)
    edge_weights = _edge_weights(coords, src)
    return edge_idx, edge_weights


# speedup vs baseline: 1.9379x; 1.0445x over previous
import functools

import jax
import jax.numpy as jnp
from jax import lax
from jax.experimental import pallas as pl
from jax.experimental.pallas import tpu as pltpu
from jax.experimental.pallas import tpu_sc as plsc

K = 32
R = 256
D = 16
SC_CHUNK = 256
PADW = 128


STRIPS = 16


def _topk_body(ct_ref, x_ref, sqc_ref, sqr_ref, idx_ref, d_ref, *, n, r):
    w = n // STRIPS
    big = jnp.int32(1 << 30)
    inf = jnp.float32(jnp.inf)
    ct = ct_ref[...]
    x = x_ref[...]
    sq = sqc_ref[...]
    sqb = sqr_ref[...]
    d = sqb + sq - 2.0 * jnp.dot(x, ct, preferred_element_type=jnp.float32)
    d_ref[...] = d

    lanes_w = jax.lax.broadcasted_iota(jnp.int32, (r, w), 1)
    cols = jax.lax.broadcasted_iota(jnp.int32, (r, K), 1)

    acc = d_ref[:, 0:w]
    amt = jnp.zeros((r, w), jnp.int32)
    for t in range(1, STRIPS):
        s = d_ref[:, t * w:(t + 1) * w]
        lt = s < acc
        amt = jnp.where(lt, t, amt)
        acc = jnp.where(lt, s, acc)
    agidx = amt * w + lanes_w

    def bstep(k, carry):
        acc, selj = carry
        m = jnp.min(acc, axis=1, keepdims=True)
        eqm = acc == m
        g0 = jnp.min(jnp.where(eqm, agidx, big), axis=1, keepdims=True)
        jc = g0 % w
        selj = jnp.where(cols == k, jc, selj)
        acc = jnp.where(lanes_w == jc, inf, acc)
        return acc, selj

    _, selj = jax.lax.fori_loop(
        0, K, bstep, (acc, jnp.zeros((r, K), jnp.int32)), unroll=False)

    n_half = max(w // 128, 1)
    jl = selj % min(w, 128)
    jh = selj // min(w, 128)
    cand_parts, gidx_parts = [], []
    for t in range(STRIPS):
        v = None
        for h in range(n_half):
            hw = min(w, 128)
            part = d_ref[:, t * w + h * hw: t * w + (h + 1) * hw]
            g = jnp.take_along_axis(part, jl, axis=1)
            v = g if v is None else jnp.where(jh == h, g, v)
        cand_parts.append(v)
        gidx_parts.append(t * w + selj)
    cand = jnp.concatenate(cand_parts, axis=1)
    gidx = jnp.concatenate(gidx_parts, axis=1)

    def dstep(k, cand):
        m = jnp.min(cand, axis=1, keepdims=True)
        eqm = cand == m
        masked_g = jnp.where(eqm, gidx, big)
        gwin = jnp.min(masked_g, axis=1, keepdims=True)
        idx_ref[...] = jnp.where(cols == k, gwin, idx_ref[...])
        return jnp.where(masked_g == gwin, inf, cand)

    jax.lax.fori_loop(0, K, dstep, cand, unroll=False)


def _nbr_topk(coords):
    n, d_feat = coords.shape
    r = R if n % R == 0 else n
    grid = (n // r,)
    body = functools.partial(_topk_body, n=n, r=r)
    sq = jnp.sum(coords * coords, axis=1)
    return pl.pallas_call(
        body,
        grid=grid,
        in_specs=[
            pl.BlockSpec((d_feat, n), lambda i: (0, 0)),
            pl.BlockSpec((r, d_feat), lambda i: (i, 0)),
            pl.BlockSpec((1, n), lambda i: (0, 0)),
            pl.BlockSpec((r, 1), lambda i: (i, 0)),
        ],
        out_specs=pl.BlockSpec((r, K), lambda i: (i, 0)),
        out_shape=jax.ShapeDtypeStruct((n, K), jnp.int32),
        scratch_shapes=[pltpu.VMEM((r, n), jnp.float32)],
    )(coords.T, coords, sq.reshape(1, n), sq.reshape(n, 1))


def _newton_sqrt(x):
    i = lax.bitcast_convert_type(x, jnp.int32)
    y = lax.bitcast_convert_type(
        jnp.int32(0x1FBD1DF5) + lax.shift_right_logical(i, 1), jnp.float32)
    for _ in range(3):
        y = 0.5 * (y + x / y)
    return y


def _edge_weights(coords, src):
    n = coords.shape[0]
    e_total = n * K
    nw = 32
    per = e_total // nw
    n_chunks = per // SC_CHUNK
    coords_pad = jnp.pad(coords, ((0, 0), (0, PADW - D)))
    mesh = plsc.VectorSubcoreMesh(core_axis_name="c", subcore_axis_name="s")

    @functools.partial(
        pl.kernel, mesh=mesh,
        out_type=jax.ShapeDtypeStruct((e_total,), jnp.float32),
        scratch_types=[
            pltpu.VMEM((2, SC_CHUNK), jnp.int32),
            pltpu.VMEM((2 * SC_CHUNK, PADW), jnp.float32),
            pltpu.VMEM((per // K, D), jnp.float32),
            pltpu.VMEM((SC_CHUNK,), jnp.float32),
            pltpu.SemaphoreType.DMA,
            pltpu.SemaphoreType.DMA,
        ],
    )
    def edge_kernel(coords_hbm, cpad_hbm, src_hbm, out_hbm, idx_v, rows_v,
                    dst_v, w_v, sem0, sem1):
        lane = lax.iota(jnp.int32, D)
        wid = lax.axis_index("s") * 2 + lax.axis_index("c")
        base = wid * per
        sems = (sem0, sem1)
        row0 = pl.multiple_of(base // K, per // K)
        pltpu.sync_copy(coords_hbm.at[pl.ds(row0, per // K)], dst_v)

        def off_of(ci):
            return pl.multiple_of(base + ci * SC_CHUNK, SC_CHUNK)

        def fetch(ci, b):
            pltpu.sync_copy(src_hbm.at[pl.ds(off_of(ci), SC_CHUNK)],
                            idx_v.at[b])
            for j in range(SC_CHUNK // 128):
                pltpu.async_copy(
                    cpad_hbm.at[idx_v.at[b, pl.ds(j * 128, 128)]],
                    rows_v.at[pl.ds(b * SC_CHUNK + j * 128, 128)], sems[b])

        def wait(b):
            for j in range(SC_CHUNK // 128):
                pltpu.make_async_copy(
                    cpad_hbm.at[idx_v.at[b, pl.ds(j * 128, 128)]],
                    rows_v.at[pl.ds(b * SC_CHUNK + j * 128, 128)],
                    sems[b]).wait()

        def compute(ci, b):
            def group(g, _):
                dv = dst_v[ci * (SC_CHUNK // K) + g // 2]
                acc = jnp.zeros((D,), jnp.float32)
                for i in range(D):
                    diff = rows_v[b * SC_CHUNK + g * D + i, pl.ds(0, D)] - dv
                    s = diff * diff
                    for sh in (1, 2, 4, 8):
                        s = s + jnp.take(s, lane ^ sh)
                    acc = jnp.where(lane == i, s, acc)
                w_v[pl.ds(g * D, D)] = _newton_sqrt(acc)
                return 0

            lax.fori_loop(0, SC_CHUNK // D, group, 0, unroll=False)
            pltpu.sync_copy(w_v, out_hbm.at[pl.ds(off_of(ci), SC_CHUNK)])

        fetch(0, 0)

        def pair(ci2, _):
            c0 = 2 * ci2
            fetch(c0 + 1, 1)
            wait(0)
            compute(c0, 0)

            @pl.when(ci2 + 1 < n_chunks // 2)
            def _():
                fetch(c0 + 2, 0)

            wait(1)
            compute(c0 + 1, 1)
            return 0

        lax.fori_loop(0, n_chunks // 2, pair, 0, unroll=False)

    return edge_kernel(coords, coords_pad, src)


def kernel(coords):
    n, _ = coords.shape
    nbr = _nbr_topk(coords)
    src = nbr.reshape(-1)
    dst = jnp.repeat(jnp.arange(n, dtype=jnp.int32), K)
    edge_idx = jnp.stack([src, dst], axis=0)
    edge_weights = _edge_weights(coords, src)
    return edge_idx, edge_weights


# speedup vs baseline: 2.3003x; 1.1870x over previous
import functools

import jax
import jax.numpy as jnp
from jax import lax
from jax.experimental import pallas as pl
from jax.experimental.pallas import tpu as pltpu
from jax.experimental.pallas import tpu_sc as plsc

K = 32
R = 512
D = 16
SC_CHUNK = 256
PADW = 128


STRIPS = 16


def _topk_body(ct_ref, x_ref, sqc_ref, sqr_ref, idx_ref, d_ref, *, n, r):
    w = n // STRIPS
    big = jnp.int32(1 << 30)
    inf = jnp.float32(jnp.inf)
    ct = ct_ref[...]
    x = x_ref[...]
    sq = sqc_ref[...]
    sqb = sqr_ref[...]
    d = sqb + sq - 2.0 * jnp.dot(x, ct, preferred_element_type=jnp.float32)
    d_ref[...] = d

    lanes_w = jax.lax.broadcasted_iota(jnp.int32, (r, w), 1)
    cols = jax.lax.broadcasted_iota(jnp.int32, (r, K), 1)

    acc = d_ref[:, 0:w]
    amt = jnp.zeros((r, w), jnp.int32)
    for t in range(1, STRIPS):
        s = d_ref[:, t * w:(t + 1) * w]
        lt = s < acc
        amt = jnp.where(lt, t, amt)
        acc = jnp.where(lt, s, acc)
    agidx = amt * w + lanes_w

    def bstep(k, carry):
        acc, selj = carry
        m = jnp.min(acc, axis=1, keepdims=True)
        eqm = acc == m
        g0 = jnp.min(jnp.where(eqm, agidx, big), axis=1, keepdims=True)
        jc = g0 % w
        selj = jnp.where(cols == k, jc, selj)
        acc = jnp.where(lanes_w == jc, inf, acc)
        return acc, selj

    _, selj = jax.lax.fori_loop(
        0, K, bstep, (acc, jnp.zeros((r, K), jnp.int32)), unroll=False)

    n_half = max(w // 128, 1)
    jl = selj % min(w, 128)
    jh = selj // min(w, 128)
    cand_parts, gidx_parts = [], []
    for t in range(STRIPS):
        v = None
        for h in range(n_half):
            hw = min(w, 128)
            part = d_ref[:, t * w + h * hw: t * w + (h + 1) * hw]
            g = jnp.take_along_axis(part, jl, axis=1)
            v = g if v is None else jnp.where(jh == h, g, v)
        cand_parts.append(v)
        gidx_parts.append(t * w + selj)
    cand = jnp.concatenate(cand_parts, axis=1)
    gidx = jnp.concatenate(gidx_parts, axis=1)

    def dstep(k, cand):
        m = jnp.min(cand, axis=1, keepdims=True)
        eqm = cand == m
        masked_g = jnp.where(eqm, gidx, big)
        gwin = jnp.min(masked_g, axis=1, keepdims=True)
        idx_ref[...] = jnp.where(cols == k, gwin, idx_ref[...])
        return jnp.where(masked_g == gwin, inf, cand)

    jax.lax.fori_loop(0, K, dstep, cand, unroll=False)


def _nbr_topk(coords):
    n, d_feat = coords.shape
    r = R if n % R == 0 else n
    grid = (n // r,)
    body = functools.partial(_topk_body, n=n, r=r)
    sq = jnp.sum(coords * coords, axis=1)
    return pl.pallas_call(
        body,
        grid=grid,
        in_specs=[
            pl.BlockSpec((d_feat, n), lambda i: (0, 0)),
            pl.BlockSpec((r, d_feat), lambda i: (i, 0)),
            pl.BlockSpec((1, n), lambda i: (0, 0)),
            pl.BlockSpec((r, 1), lambda i: (i, 0)),
        ],
        out_specs=pl.BlockSpec((r, K), lambda i: (i, 0)),
        out_shape=jax.ShapeDtypeStruct((n, K), jnp.int32),
        scratch_shapes=[pltpu.VMEM((r, n), jnp.float32)],
    )(coords.T, coords, sq.reshape(1, n), sq.reshape(n, 1))


def _newton_sqrt(x):
    i = lax.bitcast_convert_type(x, jnp.int32)
    y = lax.bitcast_convert_type(
        jnp.int32(0x1FBD1DF5) + lax.shift_right_logical(i, 1), jnp.float32)
    for _ in range(3):
        y = 0.5 * (y + x / y)
    return y


def _edge_weights(coords, src):
    n = coords.shape[0]
    e_total = n * K
    nw = 32
    per = e_total // nw
    n_chunks = per // SC_CHUNK
    coords_pad = jnp.pad(coords, ((0, 0), (0, PADW - D)))
    mesh = plsc.VectorSubcoreMesh(core_axis_name="c", subcore_axis_name="s")

    @functools.partial(
        pl.kernel, mesh=mesh,
        out_type=jax.ShapeDtypeStruct((e_total,), jnp.float32),
        scratch_types=[
            pltpu.VMEM((2, SC_CHUNK), jnp.int32),
            pltpu.VMEM((2 * SC_CHUNK, PADW), jnp.float32),
            pltpu.VMEM((per // K, D), jnp.float32),
            pltpu.VMEM((SC_CHUNK,), jnp.float32),
            pltpu.SemaphoreType.DMA,
            pltpu.SemaphoreType.DMA,
        ],
    )
    def edge_kernel(coords_hbm, cpad_hbm, src_hbm, out_hbm, idx_v, rows_v,
                    dst_v, w_v, sem0, sem1):
        lane = lax.iota(jnp.int32, D)
        wid = lax.axis_index("s") * 2 + lax.axis_index("c")
        base = wid * per
        sems = (sem0, sem1)
        row0 = pl.multiple_of(base // K, per // K)
        pltpu.sync_copy(coords_hbm.at[pl.ds(row0, per // K)], dst_v)

        def off_of(ci):
            return pl.multiple_of(base + ci * SC_CHUNK, SC_CHUNK)

        def fetch(ci, b):
            pltpu.sync_copy(src_hbm.at[pl.ds(off_of(ci), SC_CHUNK)],
                            idx_v.at[b])
            for j in range(SC_CHUNK // 128):
                pltpu.async_copy(
                    cpad_hbm.at[idx_v.at[b, pl.ds(j * 128, 128)]],
                    rows_v.at[pl.ds(b * SC_CHUNK + j * 128, 128)], sems[b])

        def wait(b):
            for j in range(SC_CHUNK // 128):
                pltpu.make_async_copy(
                    cpad_hbm.at[idx_v.at[b, pl.ds(j * 128, 128)]],
                    rows_v.at[pl.ds(b * SC_CHUNK + j * 128, 128)],
                    sems[b]).wait()

        def compute(ci, b):
            def group(g, _):
                dv = dst_v[ci * (SC_CHUNK // K) + g // 2]
                acc = jnp.zeros((D,), jnp.float32)
                for i in range(D):
                    diff = rows_v[b * SC_CHUNK + g * D + i, pl.ds(0, D)] - dv
                    s = diff * diff
                    for sh in (1, 2, 4, 8):
                        s = s + jnp.take(s, lane ^ sh)
                    acc = jnp.where(lane == i, s, acc)
                w_v[pl.ds(g * D, D)] = _newton_sqrt(acc)
                return 0

            lax.fori_loop(0, SC_CHUNK // D, group, 0, unroll=False)
            pltpu.sync_copy(w_v, out_hbm.at[pl.ds(off_of(ci), SC_CHUNK)])

        fetch(0, 0)

        def pair(ci2, _):
            c0 = 2 * ci2
            fetch(c0 + 1, 1)
            wait(0)
            compute(c0, 0)

            @pl.when(ci2 + 1 < n_chunks // 2)
            def _():
                fetch(c0 + 2, 0)

            wait(1)
            compute(c0 + 1, 1)
            return 0

        lax.fori_loop(0, n_chunks // 2, pair, 0, unroll=False)

    return edge_kernel(coords, coords_pad, src)


def kernel(coords):
    n, _ = coords.shape
    nbr = _nbr_topk(coords)
    src = nbr.reshape(-1)
    dst = jnp.repeat(jnp.arange(n, dtype=jnp.int32), K)
    edge_idx = jnp.stack([src, dst], axis=0)
    edge_weights = _edge_weights(coords, src)
    return edge_idx, edge_weights
